# baseline (speedup 1.0000x reference)
import functools

import jax
import jax.numpy as jnp
from jax import lax
from jax.experimental import pallas as pl
from jax.experimental.pallas import tpu as pltpu
from jax.experimental.pallas import tpu_sc as plsc

_VOCAB = 1000000
_EMBED = 128
_BATCH = 16384
_L = 16
_NC = 2
_NS = 16
_NW = _NC * _NS
_B_PER_W = _BATCH // _NW
_CHUNK = 128
_N_CHUNKS = _B_PER_W // _CHUNK
_GROUPS = _CHUNK // _L


def _body(tidx_hbm, cidx_hbm, tt_hbm, ct_hbm, out_hbm,
          tidx_v, cidx_v, trows_v, crows_v, out_v, sem_t, sem_c):
    wid = lax.axis_index("s") * _NC + lax.axis_index("c")
    base = wid * _B_PER_W

    def start(ch, buf):
        cb = ch * _CHUNK
        pltpu.sync_copy(tidx_hbm.at[pl.ds(base + cb, _CHUNK)],
                        tidx_v.at[buf])
        pltpu.sync_copy(cidx_hbm.at[pl.ds(base + cb, _CHUNK)],
                        cidx_v.at[buf])
        t_cp = pltpu.async_copy(tt_hbm.at[tidx_v.at[buf]],
                                trows_v.at[buf], sem_t)
        c_cp = pltpu.async_copy(ct_hbm.at[cidx_v.at[buf]],
                                crows_v.at[buf], sem_c)
        return t_cp, c_cp

    pend = start(0, 0)
    for ch in range(_N_CHUNKS):
        buf = ch % 2
        cb = ch * _CHUNK
        pend[0].wait()
        pend[1].wait()
        if ch + 1 < _N_CHUNKS:
            pend = start(ch + 1, 1 - buf)

        def group(g, carry):
            res = jnp.zeros((_L,), jnp.float32)
            lane = lax.iota(jnp.int32, _L)
            for r in range(_L):
                row = g * _L + r
                acc = (trows_v[buf, row, pl.ds(0, _L)] *
                       crows_v[buf, row, pl.ds(0, _L)])
                for i in range(1, _EMBED // _L):
                    acc = acc + (trows_v[buf, row, pl.ds(i * _L, _L)] *
                                 crows_v[buf, row, pl.ds(i * _L, _L)])
                for sh in (8, 4, 2, 1):
                    acc = acc + acc.at[lane ^ sh].get(
                        mode="promise_in_bounds")
                res = jnp.where(lane == r, acc, res)
            out_v[pl.ds(cb + g * _L, _L)] = res
            return carry

        lax.fori_loop(0, _GROUPS, group, 0)

    pltpu.sync_copy(out_v, out_hbm.at[pl.ds(base, _B_PER_W)])


@jax.jit
def kernel(target, context, target_table, context_table):
    mesh = plsc.VectorSubcoreMesh(core_axis_name="c", subcore_axis_name="s")
    run = pl.kernel(
        _body,
        mesh=mesh,
        out_type=jax.ShapeDtypeStruct((_BATCH,), jnp.float32),
        scratch_types=[
            pltpu.VMEM((2, _CHUNK), jnp.int32),
            pltpu.VMEM((2, _CHUNK), jnp.int32),
            pltpu.VMEM((2, _CHUNK, _EMBED), jnp.float32),
            pltpu.VMEM((2, _CHUNK, _EMBED), jnp.float32),
            pltpu.VMEM((_B_PER_W,), jnp.float32),
            pltpu.SemaphoreType.DMA,
            pltpu.SemaphoreType.DMA,
        ],
    )
    flat = run(target.astype(jnp.int32), context.astype(jnp.int32),
               target_table, context_table)
    return flat.reshape(_BATCH, 1)


# speedup vs baseline: 1.1277x; 1.1277x over previous
import jax
import jax.numpy as jnp
from jax import lax
from jax.experimental import pallas as pl
from jax.experimental.pallas import tpu as pltpu
from jax.experimental.pallas import tpu_sc as plsc

_VOCAB = 1000000
_EMBED = 128
_BATCH = 16384
_L = 16
_NC = 2
_NS = 16
_NW = _NC * _NS
_B_PER_W = _BATCH // _NW
_CHUNK = 64
_N_CHUNKS = _B_PER_W // _CHUNK
_GROUPS = _CHUNK // _L


def _body(tidx_hbm, cidx_hbm, tt_hbm, ct_hbm, out_hbm,
          tidx_v, cidx_v, trows_v, crows_v, out_v,
          sem_t0, sem_t1, sem_c0, sem_c1):
    wid = lax.axis_index("s") * _NC + lax.axis_index("c")
    base = wid * _B_PER_W
    sem_t = (sem_t0, sem_t1)
    sem_c = (sem_c0, sem_c1)

    pltpu.sync_copy(tidx_hbm.at[pl.ds(base, _B_PER_W)], tidx_v)
    pltpu.sync_copy(cidx_hbm.at[pl.ds(base, _B_PER_W)], cidx_v)

    def t_copy(ch, buf):
        return pltpu.make_async_copy(
            tt_hbm.at[tidx_v.at[pl.ds(ch * _CHUNK, _CHUNK)]],
            trows_v.at[buf], sem_t[buf])

    def c_copy(ch, buf):
        return pltpu.make_async_copy(
            ct_hbm.at[cidx_v.at[pl.ds(ch * _CHUNK, _CHUNK)]],
            crows_v.at[buf], sem_c[buf])

    def start_pair(ch, buf):
        t_copy(ch, buf).start()
        c_copy(ch, buf).start()

    def wait_pair(buf):
        t_copy(0, buf).wait()
        c_copy(0, buf).wait()

    lane = lax.iota(jnp.int32, _L)
    perms = [lane ^ (1 << s) for s in range(4)]

    def rowdot(buf, row):
        prods = [trows_v[buf, row, pl.ds(i * _L, _L)] *
                 crows_v[buf, row, pl.ds(i * _L, _L)]
                 for i in range(_EMBED // _L)]
        while len(prods) > 1:
            prods = [prods[i] + prods[i + 1]
                     for i in range(0, len(prods), 2)]
        return prods[0]

    def do_chunk(buf, cb):
        for g in range(_GROUPS):

            @pl.loop(0, _L, init_carry=jnp.zeros((_L,), jnp.float32),
                     unroll=4)
            def rows(r, res):
                acc = rowdot(buf, g * _L + r)
                for s in range(4):
                    acc = acc + acc.at[perms[s]].get(
                        mode="promise_in_bounds")
                return jnp.where(lane == r, acc, res)

            out_v[pl.ds(cb + g * _L, _L)] = rows

    start_pair(0, 0)
    start_pair(1, 1)
    for ch in range(_N_CHUNKS):
        buf = ch % 2
        wait_pair(buf)
        do_chunk(buf, ch * _CHUNK)
        if ch + 2 < _N_CHUNKS:
            start_pair(ch + 2, buf)

    pltpu.sync_copy(out_v, out_hbm.at[pl.ds(base, _B_PER_W)])


@jax.jit
def kernel(target, context, target_table, context_table):
    mesh = plsc.VectorSubcoreMesh(core_axis_name="c", subcore_axis_name="s")
    run = pl.kernel(
        _body,
        mesh=mesh,
        out_type=jax.ShapeDtypeStruct((_BATCH,), jnp.float32),
        scratch_types=[
            pltpu.VMEM((_B_PER_W,), jnp.int32),
            pltpu.VMEM((_B_PER_W,), jnp.int32),
            pltpu.VMEM((2, _CHUNK, _EMBED), jnp.float32),
            pltpu.VMEM((2, _CHUNK, _EMBED), jnp.float32),
            pltpu.VMEM((_B_PER_W,), jnp.float32),
            pltpu.SemaphoreType.DMA,
            pltpu.SemaphoreType.DMA,
            pltpu.SemaphoreType.DMA,
            pltpu.SemaphoreType.DMA,
        ],
    )
    flat = run(target.astype(jnp.int32), context.astype(jnp.int32),
               target_table, context_table)
    return flat.reshape(_BATCH, 1)
